# baseline (speedup 1.0000x reference)
import functools
import jax
import jax.numpy as jnp
from jax import lax
from jax.experimental import pallas as pl
from jax.experimental.pallas import tpu as pltpu
from jax.experimental.pallas import tpu_sc as plsc

_IDX_BITS = 9
_KEY_MASK = -(1 << _IDX_BITS)
_BIAS = 1 << 23

_NW = 32
_C = 16


def _nn_body(xt_ref, xs_ref, fs_ref, w1a_ref, idx_ref, wts_ref, g_ref):
    n_t = xt_ref.shape[1]
    n_s = xs_ref.shape[2]
    b = pl.program_id(0)

    d2 = jnp.zeros((n_t, n_s), jnp.float32)
    for d in range(3):
        tcol = xt_ref[0, :, d:d + 1]
        srow = xs_ref[0, d:d + 1, :]
        diff = tcol - srow
        d2 = d2 + diff * diff

    s_iota = jax.lax.broadcasted_iota(jnp.int32, (n_t, n_s), 1)
    keyi = ((jax.lax.bitcast_convert_type(d2, jnp.int32)
             + (1 << (_IDX_BITS - 1))) & _KEY_MASK) | s_iota
    keyf = jax.lax.bitcast_convert_type(keyi + _BIAS, jnp.float32)

    masked = keyf
    mks = []
    for r in range(3):
        mk = jnp.min(masked, axis=1, keepdims=True)
        mks.append(mk)
        if r < 2:
            masked = jnp.where(masked == mk, jnp.inf, masked)

    rs = []
    idxs = []
    for mk in mks:
        bits = jax.lax.bitcast_convert_type(mk, jnp.int32) - _BIAS
        idxs.append((bits & ((1 << _IDX_BITS) - 1)) + b * n_s)
        d2k = jax.lax.bitcast_convert_type(bits & _KEY_MASK, jnp.float32)
        rs.append(jax.lax.rsqrt(jnp.maximum(d2k, 1e-20)))
    norm = rs[0] + rs[1] + rs[2]
    inv = 1.0 / (norm * (1.0 + 1e-6))
    cs = [r * inv for r in rs]

    idx_ref[0] = jnp.concatenate(idxs, axis=1)
    wts_ref[0] = jnp.concatenate(
        cs + [jnp.zeros((n_t, 13), jnp.float32)], axis=1)
    g_ref[0] = jnp.dot(fs_ref[0], w1a_ref[...],
                       preferred_element_type=jnp.float32)


def _mlp_body(acc_ref, ft_ref, w1b_ref, w2_ref, out_ref):
    h = acc_ref[0] + jnp.dot(ft_ref[0], w1b_ref[...],
                             preferred_element_type=jnp.float32)
    h = jnp.maximum(h, 0.0)
    out = jnp.dot(h, w2_ref[...], preferred_element_type=jnp.float32)
    out_ref[0] = jnp.maximum(out, 0.0)


def _sc_gather(idx_hbm, wts_hbm, g_hbm, out_hbm,
               idx_v, wts_v, rows0, rows1, ob0, ob1,
               sem_r0, sem_r1, sem_o0, sem_o1):
    n_tot = out_hbm.shape[0]
    d_out = out_hbm.shape[1]
    pw = n_tot // _NW
    nc = pw // _C
    wid = lax.axis_index("s") * 2 + lax.axis_index("c")
    base = wid * pw

    pltpu.sync_copy(idx_hbm.at[pl.ds(base * 3, pw * 3)], idx_v)
    pltpu.sync_copy(wts_hbm.at[pl.ds(base, pw)], wts_v)

    def gather(ci, rows, sem):
        return pltpu.async_copy(
            g_hbm.at[idx_v.at[pl.ds(ci * (3 * _C), 3 * _C)]], rows, sem)

    def gather_wait(ci, rows, sem):
        pltpu.make_async_copy(
            g_hbm.at[idx_v.at[pl.ds(ci * (3 * _C), 3 * _C)]], rows,
            sem).wait()

    gather(0, rows0, sem_r0)
    gather(1, rows1, sem_r1)

    def compute(ci, rows, ob):
        toff = ci * _C

        def tgt(i, _):
            wv = wts_v[toff + i]
            w0 = wv[0]
            w1 = wv[1]
            w2 = wv[2]
            for v in range(d_out // 16):
                sl = pl.ds(v * 16, 16)
                a = rows[3 * i, sl] * w0
                a = a + rows[3 * i + 1, sl] * w1
                a = a + rows[3 * i + 2, sl] * w2
                ob[i, sl] = a
            return 0

        lax.fori_loop(0, _C, tgt, 0)

    def phase(j, ci, rows, sem_r, ob, sem_o):
        gather_wait(ci, rows, sem_r)

        @pl.when(j > 0)
        def _():
            pltpu.make_async_copy(ob, out_hbm.at[pl.ds(base, _C)],
                                  sem_o).wait()

        compute(ci, rows, ob)
        pltpu.async_copy(ob, out_hbm.at[pl.ds(base + ci * _C, _C)], sem_o)

        @pl.when(ci + 2 <= nc - 1)
        def _():
            gather(ci + 2, rows, sem_r)

    def pair(j, _):
        phase(j, 2 * j, rows0, sem_r0, ob0, sem_o0)
        phase(j, 2 * j + 1, rows1, sem_r1, ob1, sem_o1)
        return 0

    lax.fori_loop(0, nc // 2, pair, 0)
    pltpu.make_async_copy(ob0, out_hbm.at[pl.ds(base, _C)], sem_o0).wait()
    pltpu.make_async_copy(ob1, out_hbm.at[pl.ds(base, _C)], sem_o1).wait()


@jax.jit
def kernel(xyz_target, xyz_source, feats_target, feats_source, W1, W2):
    bs, n_t, _ = xyz_target.shape
    n_s = xyz_source.shape[1]
    c_t = feats_target.shape[2]
    c_s = feats_source.shape[2]
    c_mid = W1.shape[1]

    xs = jnp.transpose(xyz_source, (0, 2, 1))
    W1a = W1[:c_s]
    W1b = W1[c_s:]

    idx3, wts3, g = pl.pallas_call(
        _nn_body,
        grid=(bs,),
        in_specs=[
            pl.BlockSpec((1, n_t, 3), lambda b: (b, 0, 0)),
            pl.BlockSpec((1, 3, n_s), lambda b: (b, 0, 0)),
            pl.BlockSpec((1, n_s, c_s), lambda b: (b, 0, 0)),
            pl.BlockSpec((c_s, c_mid), lambda b: (0, 0)),
        ],
        out_specs=[
            pl.BlockSpec((1, n_t, 3), lambda b: (b, 0, 0)),
            pl.BlockSpec((1, n_t, 16), lambda b: (b, 0, 0)),
            pl.BlockSpec((1, n_s, c_mid), lambda b: (b, 0, 0)),
        ],
        out_shape=[
            jax.ShapeDtypeStruct((bs, n_t, 3), jnp.int32),
            jax.ShapeDtypeStruct((bs, n_t, 16), jnp.float32),
            jax.ShapeDtypeStruct((bs, n_s, c_mid), jnp.float32),
        ],
    )(xyz_target, xs, feats_source, W1a)

    idx_flat = idx3.reshape(bs * n_t * 3)
    wts_flat = wts3.reshape(bs * n_t, 16)
    g_flat = g.reshape(bs * n_s, c_mid)

    sc = pl.kernel(
        _sc_gather,
        out_type=jax.ShapeDtypeStruct((bs * n_t, c_mid), jnp.float32),
        mesh=plsc.VectorSubcoreMesh(core_axis_name="c", subcore_axis_name="s"),
        scratch_types=[
            pltpu.VMEM((bs * n_t * 3 // _NW,), jnp.int32),
            pltpu.VMEM((bs * n_t // _NW, 16), jnp.float32),
            pltpu.VMEM((3 * _C, c_mid), jnp.float32),
            pltpu.VMEM((3 * _C, c_mid), jnp.float32),
            pltpu.VMEM((_C, c_mid), jnp.float32),
            pltpu.VMEM((_C, c_mid), jnp.float32),
            pltpu.SemaphoreType.DMA,
            pltpu.SemaphoreType.DMA,
            pltpu.SemaphoreType.DMA,
            pltpu.SemaphoreType.DMA,
        ],
    )
    acc = sc(idx_flat, wts_flat, g_flat)
    acc = acc.reshape(bs, n_t, c_mid)

    out = pl.pallas_call(
        _mlp_body,
        grid=(bs,),
        in_specs=[
            pl.BlockSpec((1, n_t, c_mid), lambda b: (b, 0, 0)),
            pl.BlockSpec((1, n_t, c_t), lambda b: (b, 0, 0)),
            pl.BlockSpec((c_t, c_mid), lambda b: (0, 0)),
            pl.BlockSpec(W2.shape, lambda b: (0, 0)),
        ],
        out_specs=pl.BlockSpec((1, n_t, W2.shape[1]), lambda b: (b, 0, 0)),
        out_shape=jax.ShapeDtypeStruct((bs, n_t, W2.shape[1]), jnp.float32),
    )(acc, feats_target, W1b, W2)
    return out


# speedup vs baseline: 2.6974x; 2.6974x over previous
import functools
import jax
import jax.numpy as jnp
from jax.experimental import pallas as pl
from jax.experimental.pallas import tpu as pltpu

_IDX_BITS = 9
_KEY_MASK = -(1 << _IDX_BITS)
_BIAS = 1 << 23


def _fp_body(xt_ref, xs_ref, ft_ref, fs_ref, w1a_ref, w1b_ref, w2_ref,
             out_ref):
    n_t = xt_ref.shape[1]
    n_s = xs_ref.shape[2]

    d2 = jnp.zeros((n_t, n_s), jnp.float32)
    for d in range(3):
        tcol = xt_ref[0, :, d:d + 1]
        srow = xs_ref[0, d:d + 1, :]
        diff = tcol - srow
        d2 = d2 + diff * diff

    s_iota = jax.lax.broadcasted_iota(jnp.int32, (n_t, n_s), 1)
    keyi = ((jax.lax.bitcast_convert_type(d2, jnp.int32)
             + (1 << (_IDX_BITS - 1))) & _KEY_MASK) | s_iota
    keyf = jax.lax.bitcast_convert_type(keyi + _BIAS, jnp.float32)

    masked = keyf
    mks = []
    for r in range(3):
        mk = jnp.min(masked, axis=1, keepdims=True)
        mks.append(mk)
        if r < 2:
            masked = jnp.where(masked == mk, jnp.inf, masked)

    rs = []
    for mk in mks:
        bits = jax.lax.bitcast_convert_type(mk, jnp.int32) - _BIAS
        d2k = jax.lax.bitcast_convert_type(bits & _KEY_MASK, jnp.float32)
        rs.append(jax.lax.rsqrt(jnp.maximum(d2k, 1e-20)))
    norm = rs[0] + rs[1] + rs[2]
    inv = 1.0 / (norm * (1.0 + 1e-6))
    cs = [r * inv for r in rs]

    zero = jnp.zeros((), jnp.float32)
    coeff = jnp.where(
        keyf == mks[0], cs[0],
        jnp.where(keyf == mks[1], cs[1],
                  jnp.where(keyf == mks[2], cs[2], zero)))

    g = jnp.dot(fs_ref[0], w1a_ref[...], preferred_element_type=jnp.float32)
    h = jnp.dot(coeff, g, preferred_element_type=jnp.float32)
    h = h + jnp.dot(ft_ref[0], w1b_ref[...],
                    preferred_element_type=jnp.float32)
    h = jnp.maximum(h, 0.0)
    out = jnp.dot(h, w2_ref[...], preferred_element_type=jnp.float32)
    out_ref[0] = jnp.maximum(out, 0.0)


@jax.jit
def kernel(xyz_target, xyz_source, feats_target, feats_source, W1, W2):
    bs, n_t, _ = xyz_target.shape
    n_s = xyz_source.shape[1]
    c_t = feats_target.shape[2]
    c_s = feats_source.shape[2]

    xs = jnp.transpose(xyz_source, (0, 2, 1))
    W1a = W1[:c_s]
    W1b = W1[c_s:]

    grid = (bs,)
    out = pl.pallas_call(
        _fp_body,
        grid=grid,
        compiler_params=pltpu.CompilerParams(
            dimension_semantics=("parallel",)),
        in_specs=[
            pl.BlockSpec((1, n_t, 3), lambda b: (b, 0, 0)),
            pl.BlockSpec((1, 3, n_s), lambda b: (b, 0, 0)),
            pl.BlockSpec((1, n_t, c_t), lambda b: (b, 0, 0)),
            pl.BlockSpec((1, n_s, c_s), lambda b: (b, 0, 0)),
            pl.BlockSpec((c_s, W1.shape[1]), lambda b: (0, 0)),
            pl.BlockSpec((c_t, W1.shape[1]), lambda b: (0, 0)),
            pl.BlockSpec(W2.shape, lambda b: (0, 0)),
        ],
        out_specs=pl.BlockSpec((1, n_t, W2.shape[1]), lambda b: (b, 0, 0)),
        out_shape=jax.ShapeDtypeStruct((bs, n_t, W2.shape[1]), jnp.float32),
    )(xyz_target, xs, feats_target, feats_source, W1a, W1b, W2)
    return out
